# baseline (speedup 1.0000x reference)
import jax
import jax.numpy as jnp
from jax.experimental import pallas as pl
from jax.experimental.pallas import tpu as pltpu

N = 8192
D = 64
BC = 256
NB = N // BC
NEG_SLOPE = 0.01


def _gc_kernel(adj_ref, embs_ref, w_ref, out_ref, s_ref, nm_ref, y_ref):
    p = pl.program_id(0)
    j = pl.program_id(1)

    @pl.when(jnp.logical_and(p == 0, j == 0))
    def _init():
        x = embs_ref[...]
        rn = jnp.sqrt(jnp.sum(x * x, axis=1, keepdims=True))
        s_ref[...] = jnp.sum(x, axis=1, keepdims=True) / jnp.maximum(rn, 1e-12)
        nm_ref[...] = jnp.zeros_like(nm_ref)

    a = adj_ref[...]

    @pl.when(p == 0)
    def _phase0():
        ones = jnp.ones((N, 1), dtype=jnp.float32)
        csq = jax.lax.dot_general(
            a * a, ones, (((0,), (0,)), ((), ())),
            preferred_element_type=jnp.float32,
        )
        s_blk = s_ref[pl.ds(j * BC, BC), :]
        t_blk = s_blk / jnp.maximum(jnp.sqrt(csq), 1e-12)
        nm_ref[...] += jnp.dot(a, t_blk, preferred_element_type=jnp.float32)

    @pl.when(jnp.logical_and(p == 1, j == 0))
    def _mid():
        total = jnp.sum(nm_ref[...])
        nm = nm_ref[...] * (1.0 / total)
        h = jnp.dot(embs_ref[...], w_ref[...], preferred_element_type=jnp.float32)
        y_ref[...] = nm * h

    @pl.when(p == 1)
    def _phase1():
        y_blk = y_ref[pl.ds(j * BC, BC), :]
        contrib = jnp.dot(a, y_blk, preferred_element_type=jnp.float32)

        @pl.when(j == 0)
        def _first():
            out_ref[...] = contrib

        @pl.when(j > 0)
        def _rest():
            out_ref[...] += contrib

        @pl.when(j == NB - 1)
        def _last():
            o = out_ref[...]
            out_ref[...] = jnp.where(o >= 0, o, NEG_SLOPE * o)


@jax.jit
def _run(adj_matrix, node_embs, W):
    return pl.pallas_call(
        _gc_kernel,
        grid=(2, NB),
        in_specs=[
            pl.BlockSpec((N, BC), lambda p, j: (0, j)),
            pl.BlockSpec((N, D), lambda p, j: (0, 0)),
            pl.BlockSpec((D, D), lambda p, j: (0, 0)),
        ],
        out_specs=pl.BlockSpec((N, D), lambda p, j: (0, 0)),
        out_shape=jax.ShapeDtypeStruct((N, D), jnp.float32),
        scratch_shapes=[
            pltpu.VMEM((N, 1), jnp.float32),
            pltpu.VMEM((N, 1), jnp.float32),
            pltpu.VMEM((N, D), jnp.float32),
        ],
        compiler_params=pltpu.CompilerParams(
            dimension_semantics=("arbitrary", "arbitrary"),
        ),
    )(adj_matrix, node_embs, W)


def kernel(adj_matrix, node_embs, W):
    return _run(adj_matrix, node_embs, W)


# speedup vs baseline: 1.5162x; 1.5162x over previous
import jax
import jax.numpy as jnp
from jax.experimental import pallas as pl
from jax.experimental.pallas import tpu as pltpu

N = 8192
D = 64
BCA = 256
BRB = 512
NBA = N // BCA
NBB = N // BRB
NEG_SLOPE = 0.01


def _pass_a(adj_ref, embs_t_ref, w_ref, y_ref, acc_ref, sr_ref):
    j = pl.program_id(0)

    @pl.when(j == 0)
    def _init():
        xt = embs_t_ref[...]
        rn = jnp.sqrt(jnp.sum(xt * xt, axis=0, keepdims=True))
        sr_ref[...] = jnp.sum(xt, axis=0, keepdims=True) / jnp.maximum(rn, 1e-12)
        acc_ref[...] = jnp.zeros_like(acc_ref)

    a = adj_ref[...]
    csq = jnp.sum(a * a, axis=0, keepdims=True)
    s_blk = sr_ref[:, pl.ds(j * BCA, BCA)]
    t_row = s_blk / jnp.maximum(jnp.sqrt(csq), 1e-12)

    acc = acc_ref[...]
    for k in range(BCA // 128):
        acc = acc + a[:, k * 128:(k + 1) * 128] * t_row[:, k * 128:(k + 1) * 128]
    acc_ref[...] = acc

    @pl.when(j == NBA - 1)
    def _finalize():
        nm = jnp.sum(acc, axis=1, keepdims=True)
        total = jnp.sum(nm)
        h = jax.lax.dot_general(
            embs_t_ref[...], w_ref[...], (((0,), (0,)), ((), ())),
            preferred_element_type=jnp.float32,
        )
        y_ref[...] = (nm * (1.0 / total)) * h


def _pass_b(adj_ref, y_ref, out_ref):
    o = jnp.dot(adj_ref[...], y_ref[...], preferred_element_type=jnp.float32)
    out_ref[...] = jnp.where(o >= 0, o, NEG_SLOPE * o)


@jax.jit
def _run(adj_matrix, node_embs, W):
    embs_t = node_embs.T

    y = pl.pallas_call(
        _pass_a,
        grid=(NBA,),
        in_specs=[
            pl.BlockSpec((N, BCA), lambda j: (0, j)),
            pl.BlockSpec((D, N), lambda j: (0, 0)),
            pl.BlockSpec((D, D), lambda j: (0, 0)),
        ],
        out_specs=pl.BlockSpec((N, D), lambda j: (0, 0)),
        out_shape=jax.ShapeDtypeStruct((N, D), jnp.float32),
        scratch_shapes=[
            pltpu.VMEM((N, 128), jnp.float32),
            pltpu.VMEM((1, N), jnp.float32),
        ],
        compiler_params=pltpu.CompilerParams(
            dimension_semantics=("arbitrary",),
        ),
    )(adj_matrix, embs_t, W)

    out = pl.pallas_call(
        _pass_b,
        grid=(NBB,),
        in_specs=[
            pl.BlockSpec((BRB, N), lambda i: (i, 0)),
            pl.BlockSpec((N, D), lambda i: (0, 0)),
        ],
        out_specs=pl.BlockSpec((BRB, D), lambda i: (i, 0)),
        out_shape=jax.ShapeDtypeStruct((N, D), jnp.float32),
        compiler_params=pltpu.CompilerParams(
            dimension_semantics=("arbitrary",),
        ),
    )(adj_matrix, y)
    return out


def kernel(adj_matrix, node_embs, W):
    return _run(adj_matrix, node_embs, W)
